# Initial kernel scaffold; baseline (speedup 1.0000x reference)
import jax
import jax.numpy as jnp
from jax.experimental import pallas as pl


def kernel(path):
    raise NotImplementedError("write your pallas kernel here")


# speedup vs baseline: 4.8449x; 4.8449x over previous
import jax
import jax.numpy as jnp
from jax import lax
from jax.experimental import pallas as pl
from jax.experimental.pallas import tpu as pltpu

_C = 8


def _sig_kernel(p_ref, o1, o2, o3, o4):
    steps = p_ref.shape[0]
    B = p_ref.shape[1]
    f32 = jnp.float32

    def expander(m):
        col = lax.broadcasted_iota(jnp.int32, (_C, _C * m), 1) // m
        row = lax.broadcasted_iota(jnp.int32, (_C, _C * m), 0)
        return (col == row).astype(f32)

    e64 = expander(8)
    e512 = expander(64)
    e4096 = expander(512)

    x0 = p_ref[0]
    init = (
        x0,
        jnp.zeros((B, 8), f32),
        jnp.zeros((B, 64), f32),
        jnp.zeros((B, 512), f32),
        jnp.zeros((B, 4096), f32),
    )

    def step(t, carry):
        xprev, s1, s2, s3, s4 = carry
        x = p_ref[t]
        dx = x - xprev
        r64 = jnp.dot(dx, e64, preferred_element_type=f32)
        r512 = jnp.dot(dx, e512, preferred_element_type=f32)
        r4096 = jnp.dot(dx, e4096, preferred_element_type=f32)

        def ot2(c):
            return r64 * pltpu.repeat(c, 8, axis=1)

        def ot3(d):
            return r512 * pltpu.repeat(d, 8, axis=1)

        def ot4(e):
            return r4096 * pltpu.repeat(e, 8, axis=1)

        s4n = s4 + ot4(s3 + 0.5 * ot3(s2 + (1.0 / 3.0) * ot2(s1 + 0.25 * dx)))
        s3n = s3 + ot3(s2 + 0.5 * ot2(s1 + (1.0 / 3.0) * dx))
        s2n = s2 + ot2(s1 + 0.5 * dx)
        s1n = s1 + dx
        return (x, s1n, s2n, s3n, s4n)

    _, s1, s2, s3, s4 = lax.fori_loop(1, steps, step, init)
    o1[...] = s1
    o2[...] = s2
    o3[...] = s3
    o4[...] = s4


def kernel(path):
    n, length, c = path.shape
    pt = jnp.swapaxes(path, 0, 1)
    grid_n = 2
    B = n // grid_n
    out_shape = tuple(
        jax.ShapeDtypeStruct((n, c**k), jnp.float32) for k in range(1, 5)
    )
    s1, s2r, s3r, s4r = pl.pallas_call(
        _sig_kernel,
        grid=(grid_n,),
        in_specs=[pl.BlockSpec((length, B, c), lambda i: (0, i, 0))],
        out_specs=tuple(
            pl.BlockSpec((B, c**k), lambda i: (i, 0)) for k in range(1, 5)
        ),
        out_shape=out_shape,
        compiler_params=pltpu.CompilerParams(
            dimension_semantics=("parallel",),
        ),
        name="signature_scan",
    )(pt)
    s2 = s2r.reshape(n, 8, 8).transpose(0, 2, 1).reshape(n, 64)
    s3 = s3r.reshape(n, 8, 8, 8).transpose(0, 3, 2, 1).reshape(n, 512)
    s4 = s4r.reshape(n, 8, 8, 8, 8).transpose(0, 4, 3, 2, 1).reshape(n, 4096)
    return jnp.concatenate([s1, s2, s3, s4], axis=-1)


# speedup vs baseline: 9.1547x; 1.8895x over previous
import jax
import jax.numpy as jnp
from jax import lax
from jax.experimental import pallas as pl
from jax.experimental.pallas import tpu as pltpu

_C = 8
_W = 512


def _sig_kernel(p_ref, o1, o2, o3, o4):
    steps = p_ref.shape[0]
    B = p_ref.shape[1]
    f32 = jnp.float32

    lane = lax.broadcasted_iota(jnp.int32, (_C, _W), 1)
    row = lax.broadcasted_iota(jnp.int32, (_C, _W), 0)
    e1 = ((lane & 7) == row).astype(f32)
    e2 = (((lane >> 3) & 7) == row).astype(f32)
    e3 = ((lane >> 6) == row).astype(f32)

    o3[...] = jnp.zeros((B, _W), f32)
    o4[...] = jnp.zeros((B, _C * _W), f32)

    def pats(dx):
        d1 = jnp.dot(dx, e1, preferred_element_type=f32)
        d2 = jnp.dot(dx, e2, preferred_element_type=f32)
        d3 = jnp.dot(dx, e3, preferred_element_type=f32)
        return d1, d2, d3

    x0 = p_ref[0]
    x1 = p_ref[1]
    dx1 = x1 - x0
    d11, d21, d31 = pats(dx1)
    init = (
        x1,
        dx1,
        d11,
        d21,
        d31,
        jnp.zeros((B, _W), f32),
        jnp.zeros((B, _W), f32),
    )

    def step(t, carry):
        x, dx, d1, d2, d3, s1, s2 = carry
        xn = p_ref[jnp.minimum(t + 1, steps - 1)]
        dxn = xn - x
        d1n, d2n, d3n = pats(dxn)

        s3v = o3[...]
        ct = s1 + 0.25 * d1
        gt = s2 + (1.0 / 3.0) * (d2 * ct)
        h = s3v + 0.5 * (d3 * gt)
        for j in range(_C):
            o4[:, _W * j : _W * (j + 1)] += dx[:, j : j + 1] * h
        cv = s1 + (1.0 / 3.0) * d1
        dv = s2 + 0.5 * (d2 * cv)
        o3[...] = s3v + d3 * dv
        av = s1 + 0.5 * d1
        s2n = s2 + d2 * av
        s1n = s1 + d1
        return (xn, dxn, d1n, d2n, d3n, s1n, s2n)

    carry = lax.fori_loop(1, steps, step, init)
    s1, s2 = carry[5], carry[6]
    o1[...] = s1
    o2[...] = s2


def kernel(path):
    n, length, c = path.shape
    pt = jnp.swapaxes(path, 0, 1)
    grid_n = 2
    B = n // grid_n
    out_shape = (
        jax.ShapeDtypeStruct((n, _W), jnp.float32),
        jax.ShapeDtypeStruct((n, _W), jnp.float32),
        jax.ShapeDtypeStruct((n, _W), jnp.float32),
        jax.ShapeDtypeStruct((n, _C * _W), jnp.float32),
    )
    s1t, s2t, s3r, s4r = pl.pallas_call(
        _sig_kernel,
        grid=(grid_n,),
        in_specs=[pl.BlockSpec((length, B, c), lambda i: (0, i, 0))],
        out_specs=(
            pl.BlockSpec((B, _W), lambda i: (i, 0)),
            pl.BlockSpec((B, _W), lambda i: (i, 0)),
            pl.BlockSpec((B, _W), lambda i: (i, 0)),
            pl.BlockSpec((B, _C * _W), lambda i: (i, 0)),
        ),
        out_shape=out_shape,
        compiler_params=pltpu.CompilerParams(
            dimension_semantics=("parallel",),
        ),
        name="signature_scan",
    )(pt)
    s1 = s1t[:, :8]
    s2 = s2t[:, :64].reshape(n, 8, 8).transpose(0, 2, 1).reshape(n, 64)
    s3 = s3r.reshape(n, 8, 8, 8).transpose(0, 3, 2, 1).reshape(n, 512)
    s4 = s4r.reshape(n, 8, 8, 8, 8).transpose(0, 4, 3, 2, 1).reshape(n, 4096)
    return jnp.concatenate([s1, s2, s3, s4], axis=-1)
